# baseline (speedup 1.0000x reference)
import functools

import jax
import jax.numpy as jnp
from jax import lax
from jax.experimental import pallas as pl
from jax.experimental.pallas import tpu as pltpu
from jax.experimental.pallas import tpu_sc as plsc

N = 10000
E = 320000
F = 128
H = 128
R = 4
L = 12

NC = 2
NS = 16
NW = NC * NS
CHUNK = 128
NCHUNK = 80
EPT = NCHUNK * CHUNK
E_PAD = EPT * NW
ROWS_PT = N // NS
ACC_ROWS = N + 8


def _sc_body(allh_hbm, packed_hbm, zeros_hbm, out_hbm,
             pbuf, gc0, gc1, dc0, dc1, rows0, rows1, acc, sem0, sem1):
    c = lax.axis_index("c")
    s = lax.axis_index("s")
    wid = c * NS + s

    pltpu.sync_copy(zeros_hbm, acc.at[pl.ds(s * ROWS_PT, ROWS_PT)])

    pltpu.sync_copy(packed_hbm.at[wid], pbuf)
    plsc.subcore_barrier()

    def unpack(j, gc, dc):
        for k in range(CHUNK // 16):
            v = pbuf[j, pl.ds(k * 16, 16)]
            gc[pl.ds(k * 16, 16)] = v & 0xFFFF
            dc[pl.ds(k * 16, 16)] = lax.shift_right_logical(v, 16)

    def start(rbuf, gc, sem):
        pltpu.async_copy(allh_hbm.at[gc], rbuf, sem)

    def gwait(rbuf, gc, sem):
        pltpu.make_async_copy(allh_hbm.at[gc], rbuf, sem).wait()

    def scat(rbuf, dc):
        pltpu.sync_copy(rbuf, acc.at[dc], add=True)

    def chunk(j, carry):
        unpack(j, gc0, dc0)
        start(rows0, gc0, sem0)
        gwait(rows0, gc0, sem0)
        scat(rows0, dc0)
        return carry

    lax.fori_loop(0, NCHUNK, chunk, 0)
    plsc.subcore_barrier()

    pltpu.sync_copy(acc.at[pl.ds(s * ROWS_PT, ROWS_PT)], out_hbm.at[c, s])


@functools.cache
def _sc_aggregate():
    return pl.kernel(
        _sc_body,
        out_type=jax.ShapeDtypeStruct((NC, NS, ROWS_PT, H), jnp.float32),
        mesh=plsc.VectorSubcoreMesh(core_axis_name="c", subcore_axis_name="s",
                                    num_cores=NC, num_subcores=NS),
        scratch_types=[
            pltpu.VMEM((NCHUNK, CHUNK), jnp.int32),
            pltpu.VMEM((CHUNK,), jnp.int32),
            pltpu.VMEM((CHUNK,), jnp.int32),
            pltpu.VMEM((CHUNK,), jnp.int32),
            pltpu.VMEM((CHUNK,), jnp.int32),
            pltpu.VMEM((CHUNK, H), jnp.float32),
            pltpu.VMEM((CHUNK, H), jnp.float32),
            pltpu.VMEM_SHARED((ACC_ROWS, H), jnp.float32),
            pltpu.SemaphoreType.DMA,
            pltpu.SemaphoreType.DMA,
        ],
    )


BN = 2000
_GRID = N // BN


def _dot(a, b):
    return jnp.dot(a, b, preferred_element_type=jnp.float32)


def _init_body(x_ref, win_ref, bin_ref, wrel_ref, h_ref, allh_ref):
    h = jnp.tanh(_dot(x_ref[...], win_ref[...]) + bin_ref[...])
    h_ref[...] = h
    for r in range(R):
        allh_ref[r] = _dot(h, wrel_ref[r])


def _update(h, p_ref, wself_ref, brel_ref, wu1a_ref, wu1b_ref, bu1_ref,
            wu2a_ref, wu2b_ref, bu2_ref):
    agg = p_ref[0] + p_ref[1]
    msg = jnp.tanh(agg + _dot(h, wself_ref[...]) + brel_ref[...])
    mid = jnp.tanh(_dot(h, wu1a_ref[...]) + _dot(msg, wu1b_ref[...])
                   + bu1_ref[...])
    return jnp.tanh(_dot(h, wu2a_ref[...]) + _dot(mid, wu2b_ref[...])
                    + bu2_ref[...])


def _mid_body(h_ref, p_ref, wself_ref, brel_ref, wu1a_ref, wu1b_ref, bu1_ref,
              wu2a_ref, wu2b_ref, bu2_ref, wrel_ref, hn_ref, allh_ref):
    hn = _update(h_ref[...], p_ref, wself_ref, brel_ref, wu1a_ref, wu1b_ref,
                 bu1_ref, wu2a_ref, wu2b_ref, bu2_ref)
    hn_ref[...] = hn
    for r in range(R):
        allh_ref[r] = _dot(hn, wrel_ref[r])


def _last_body(h_ref, p_ref, wself_ref, brel_ref, wu1a_ref, wu1b_ref, bu1_ref,
               wu2a_ref, wu2b_ref, bu2_ref, hn_ref):
    hn_ref[...] = _update(h_ref[...], p_ref, wself_ref, brel_ref, wu1a_ref,
                          wu1b_ref, bu1_ref, wu2a_ref, wu2b_ref, bu2_ref)


def _row_spec(width=H):
    return pl.BlockSpec((BN, width), lambda i: (i, 0))


def _full_spec(shape):
    nd = len(shape)
    return pl.BlockSpec(shape, lambda i, _n=nd: (0,) * _n)


_P_SPEC = pl.BlockSpec((NC, BN, H), lambda i: (0, i, 0))
_ALLH_SPEC = pl.BlockSpec((R, BN, H), lambda i: (0, i, 0))

_W_SPECS = [
    _full_spec((H, H)),
    _full_spec((1, H)),
    _full_spec((H, 2 * H)),
    _full_spec((H, 2 * H)),
    _full_spec((1, 2 * H)),
    _full_spec((H, H)),
    _full_spec((2 * H, H)),
    _full_spec((1, H)),
]

_H_OUT = jax.ShapeDtypeStruct((N, H), jnp.float32)
_ALLH_OUT = jax.ShapeDtypeStruct((R, N, H), jnp.float32)

_init_call = pl.pallas_call(
    _init_body,
    grid=(_GRID,),
    in_specs=[_row_spec(F), _full_spec((F, H)), _full_spec((1, H)),
              _full_spec((R, H, H))],
    out_specs=[_row_spec(), _ALLH_SPEC],
    out_shape=[_H_OUT, _ALLH_OUT],
)

_mid_call = pl.pallas_call(
    _mid_body,
    grid=(_GRID,),
    in_specs=[_row_spec(), _P_SPEC] + _W_SPECS + [_full_spec((R, H, H))],
    out_specs=[_row_spec(), _ALLH_SPEC],
    out_shape=[_H_OUT, _ALLH_OUT],
)

_last_call = pl.pallas_call(
    _last_body,
    grid=(_GRID,),
    in_specs=[_row_spec(), _P_SPEC] + _W_SPECS,
    out_specs=_row_spec(),
    out_shape=_H_OUT,
)


def kernel(x, edge_index, edge_type, W_in, b_in, W_rel, W_self, b_rel,
           W_up1, b_up1, W_up2, b_up2):
    src = edge_index[0]
    dst = edge_index[1]
    packed = jnp.left_shift(dst, 16) | (edge_type * N + src)
    packed = jnp.concatenate(
        [packed, jnp.full((E_PAD - E,), N << 16, jnp.int32)]
    ).reshape(NW, NCHUNK, CHUNK)
    zeros = jnp.zeros((ROWS_PT, H), jnp.float32)

    b_in2 = b_in.reshape(1, H)
    brel2 = b_rel.reshape(L, 1, H)
    bu12 = b_up1.reshape(L, 1, 2 * H)
    bu22 = b_up2.reshape(L, 1, H)
    wu1a = W_up1[:, :H, :]
    wu1b = W_up1[:, H:, :]
    wu2a = W_up2[:, :H, :]
    wu2b = W_up2[:, H:, :]

    h, all_h = _init_call(x, W_in, b_in2, W_rel[0])
    for l in range(L):
        partials = _sc_aggregate()(all_h.reshape(R * N, H), packed,
                                   zeros).reshape(NC, N, H)
        wargs = (W_self[l], brel2[l], wu1a[l], wu1b[l], bu12[l],
                 wu2a[l], wu2b[l], bu22[l])
        if l < L - 1:
            h, all_h = _mid_call(h, partials, *wargs, W_rel[l + 1])
        else:
            h = _last_call(h, partials, *wargs)
    return h


# speedup vs baseline: 2.9924x; 2.9924x over previous
import functools

import jax
import jax.numpy as jnp
from jax import lax
from jax.experimental import pallas as pl
from jax.experimental.pallas import tpu as pltpu
from jax.experimental.pallas import tpu_sc as plsc

N = 10000
E = 320000
F = 128
H = 128
R = 4
L = 12

NC = 2
NS = 16
NW = NC * NS
CHUNK = 125
NCHUNK = 80
EPT = NCHUNK * CHUNK
ROWS_PT = N // NS


def _sc_body(allh_hbm, gidx_hbm, dst_hbm, zeros_hbm, out_hbm,
             gbuf, dbuf, rows, acc, sem):
    c = lax.axis_index("c")
    s = lax.axis_index("s")
    wid = c * NS + s

    pltpu.sync_copy(zeros_hbm, acc.at[pl.ds(s * ROWS_PT, ROWS_PT)])

    pltpu.sync_copy(gidx_hbm.at[wid], gbuf)
    pltpu.sync_copy(dst_hbm.at[wid], dbuf)
    plsc.subcore_barrier()

    def chunk(j, carry):
        pltpu.async_copy(allh_hbm.at[gbuf.at[j]], rows, sem).wait()
        pltpu.sync_copy(rows, acc.at[dbuf.at[j]], add=True)
        return carry

    lax.fori_loop(0, NCHUNK, chunk, 0)
    plsc.subcore_barrier()

    pltpu.sync_copy(acc.at[pl.ds(s * ROWS_PT, ROWS_PT)], out_hbm.at[c, s])


@functools.cache
def _sc_aggregate():
    return pl.kernel(
        _sc_body,
        out_type=jax.ShapeDtypeStruct((NC, NS, ROWS_PT, H), jnp.float32),
        mesh=plsc.VectorSubcoreMesh(core_axis_name="c", subcore_axis_name="s",
                                    num_cores=NC, num_subcores=NS),
        scratch_types=[
            pltpu.VMEM((NCHUNK, CHUNK), jnp.int32),
            pltpu.VMEM((NCHUNK, CHUNK), jnp.int32),
            pltpu.VMEM((CHUNK, H), jnp.float32),
            pltpu.VMEM_SHARED((N, H), jnp.float32),
            pltpu.SemaphoreType.DMA,
        ],
    )


BN = 2000
_GRID = N // BN


def _dot(a, b):
    return jnp.dot(a, b, preferred_element_type=jnp.float32)


def _init_body(x_ref, win_ref, bin_ref, wrel_ref, h_ref, allh_ref):
    h = jnp.tanh(_dot(x_ref[...], win_ref[...]) + bin_ref[...])
    h_ref[...] = h
    for r in range(R):
        allh_ref[r] = _dot(h, wrel_ref[r])


def _update(h, p_ref, wself_ref, brel_ref, wu1a_ref, wu1b_ref, bu1_ref,
            wu2a_ref, wu2b_ref, bu2_ref):
    agg = p_ref[0] + p_ref[1]
    msg = jnp.tanh(agg + _dot(h, wself_ref[...]) + brel_ref[...])
    mid = jnp.tanh(_dot(h, wu1a_ref[...]) + _dot(msg, wu1b_ref[...])
                   + bu1_ref[...])
    return jnp.tanh(_dot(h, wu2a_ref[...]) + _dot(mid, wu2b_ref[...])
                    + bu2_ref[...])


def _mid_body(h_ref, p_ref, wself_ref, brel_ref, wu1a_ref, wu1b_ref, bu1_ref,
              wu2a_ref, wu2b_ref, bu2_ref, wrel_ref, hn_ref, allh_ref):
    hn = _update(h_ref[...], p_ref, wself_ref, brel_ref, wu1a_ref, wu1b_ref,
                 bu1_ref, wu2a_ref, wu2b_ref, bu2_ref)
    hn_ref[...] = hn
    for r in range(R):
        allh_ref[r] = _dot(hn, wrel_ref[r])


def _last_body(h_ref, p_ref, wself_ref, brel_ref, wu1a_ref, wu1b_ref, bu1_ref,
               wu2a_ref, wu2b_ref, bu2_ref, hn_ref):
    hn_ref[...] = _update(h_ref[...], p_ref, wself_ref, brel_ref, wu1a_ref,
                          wu1b_ref, bu1_ref, wu2a_ref, wu2b_ref, bu2_ref)


def _row_spec(width=H):
    return pl.BlockSpec((BN, width), lambda i: (i, 0))


def _full_spec(shape):
    nd = len(shape)
    return pl.BlockSpec(shape, lambda i, _n=nd: (0,) * _n)


_P_SPEC = pl.BlockSpec((NC, BN, H), lambda i: (0, i, 0))
_ALLH_SPEC = pl.BlockSpec((R, BN, H), lambda i: (0, i, 0))

_W_SPECS = [
    _full_spec((H, H)),
    _full_spec((1, H)),
    _full_spec((H, 2 * H)),
    _full_spec((H, 2 * H)),
    _full_spec((1, 2 * H)),
    _full_spec((H, H)),
    _full_spec((2 * H, H)),
    _full_spec((1, H)),
]

_H_OUT = jax.ShapeDtypeStruct((N, H), jnp.float32)
_ALLH_OUT = jax.ShapeDtypeStruct((R, N, H), jnp.float32)

_init_call = pl.pallas_call(
    _init_body,
    grid=(_GRID,),
    in_specs=[_row_spec(F), _full_spec((F, H)), _full_spec((1, H)),
              _full_spec((R, H, H))],
    out_specs=[_row_spec(), _ALLH_SPEC],
    out_shape=[_H_OUT, _ALLH_OUT],
)

_mid_call = pl.pallas_call(
    _mid_body,
    grid=(_GRID,),
    in_specs=[_row_spec(), _P_SPEC] + _W_SPECS + [_full_spec((R, H, H))],
    out_specs=[_row_spec(), _ALLH_SPEC],
    out_shape=[_H_OUT, _ALLH_OUT],
)

_last_call = pl.pallas_call(
    _last_body,
    grid=(_GRID,),
    in_specs=[_row_spec(), _P_SPEC] + _W_SPECS,
    out_specs=_row_spec(),
    out_shape=_H_OUT,
)


def kernel(x, edge_index, edge_type, W_in, b_in, W_rel, W_self, b_rel,
           W_up1, b_up1, W_up2, b_up2):
    src = edge_index[0]
    dst = edge_index[1]
    gidx = (edge_type * N + src).reshape(NW, NCHUNK, CHUNK)
    dst2 = dst.reshape(NW, NCHUNK, CHUNK)
    zeros = jnp.zeros((ROWS_PT, H), jnp.float32)

    b_in2 = b_in.reshape(1, H)
    brel2 = b_rel.reshape(L, 1, H)
    bu12 = b_up1.reshape(L, 1, 2 * H)
    bu22 = b_up2.reshape(L, 1, H)
    wu1a = W_up1[:, :H, :]
    wu1b = W_up1[:, H:, :]
    wu2a = W_up2[:, :H, :]
    wu2b = W_up2[:, H:, :]

    h, all_h = _init_call(x, W_in, b_in2, W_rel[0])
    for l in range(L):
        partials = _sc_aggregate()(all_h.reshape(R * N, H), gidx, dst2,
                                   zeros).reshape(NC, N, H)
        wargs = (W_self[l], brel2[l], wu1a[l], wu1b[l], bu12[l],
                 wu2a[l], wu2b[l], bu22[l])
        if l < L - 1:
            h, all_h = _mid_call(h, partials, *wargs, W_rel[l + 1])
        else:
            h = _last_call(h, partials, *wargs)
    return h
